# baseline (speedup 1.0000x reference)
import jax
import jax.numpy as jnp
from jax import lax
from jax.experimental import pallas as pl
from jax.experimental.pallas import tpu as pltpu
from jax.experimental.pallas import tpu_sc as plsc

_N_LAYERS = 80
_N_HEADS = 64
_INTERMEDIATE = 28672
_K_HEAD = 16
_K_INT = 7168
_K_LAYER = 16
_SCALE = 1.2

_L = 16
_INT_MIN = -(2**31)
_M31 = 0x7FFFFFFF
_NW = 32


def _lane_total(x, tmp_ref):
    for s in (8, 4, 2, 1):
        tmp_ref[pl.ds(0, _L)] = x
        tmp_ref[pl.ds(_L, _L)] = x
        x = x + tmp_ref[pl.ds(s, _L)]
    return x


def _keys_and_negcount(xv_ref, kv_ref, n, tmp_ref):
    nvec = n // _L
    zero = jnp.zeros((_L,), jnp.int32)

    def one(base, a, u):
        x = xv_ref[pl.ds(base, _L)]
        iv = lax.bitcast_convert_type(x, jnp.int32)
        sh = iv >> 31
        kv_ref[pl.ds(base, _L)] = iv ^ (sh & _M31)
        a[u % 2] = a[u % 2] + sh
        return a

    if nvec <= 8:
        a = [zero, zero]
        for j in range(nvec):
            a = one(j * _L, a, j)
        neg = a[0] + a[1]
    else:
        U = 8

        @plsc.parallel_loop(0, nvec // U, carry=(zero, zero))
        def accs(i, a):
            a = list(a)
            base = i * (U * _L)
            for u in range(U):
                a = one(base + u * _L, a, u)
            return tuple(a)

        neg = accs[0] + accs[1]
    return -_lane_total(neg, tmp_ref)[0]


def _select_threshold(kv_ref, n, k, tmp_ref, cnt_neg):
    nvec = n // _L

    def bit_body(t, st):
        p, ks, done = st

        def active(_):
            b = 31 - t
            c = p | (jnp.int32(1) << b)
            cs = c ^ _INT_MIN
            if nvec <= 16:
                acc = jnp.zeros((_L,), jnp.int32)
                for j in range(nvec):
                    kvv = kv_ref[pl.ds(j * _L, _L)]
                    acc = acc + jnp.where(kvv < cs, 1, 0)
            else:
                U = 32
                A = 8
                zero = jnp.zeros((_L,), jnp.int32)

                @plsc.parallel_loop(0, nvec // U, carry=(zero,) * A)
                def accs(i, a):
                    a = list(a)
                    base = i * (U * _L)
                    for u in range(U):
                        kvv = kv_ref[pl.ds(base + u * _L, _L)]
                        a[u % A] = a[u % A] + jnp.where(kvv < cs, 1, 0)
                    return tuple(a)

                acc = ((accs[0] + accs[1]) + (accs[2] + accs[3])) + (
                    (accs[4] + accs[5]) + (accs[6] + accs[7])
                )
            cnt = _lane_total(acc, tmp_ref)[0]
            return (
                jnp.where(cnt >= k, p, c),
                jnp.where(cnt == k, cs - 1, ks),
                cnt == k,
            )

        def idle(_):
            return p, ks, done

        return lax.cond(jnp.logical_not(done), active, idle, 0)

    p0 = jnp.where(cnt_neg >= k, jnp.int32(0), jnp.int32(_INT_MIN))
    done0 = cnt_neg == k
    ks0 = jnp.where(done0, jnp.int32(-1), jnp.int32(0))
    p, ks, done = lax.fori_loop(1, 32, bit_body, (p0, ks0, done0))
    return jnp.where(done, ks, p ^ _INT_MIN)


def _apply_mask(xv_ref, kv_ref, n, ks):
    nvec = n // _L

    def one(j_base):
        x = xv_ref[pl.ds(j_base, _L)]
        kvv = kv_ref[pl.ds(j_base, _L)]
        s = 1.0 / (1.0 + jnp.exp(x * jnp.float32(-_SCALE)))
        xv_ref[pl.ds(j_base, _L)] = jnp.where(kvv <= ks, jnp.float32(0.0), s)

    if nvec <= 8:
        for j in range(nvec):
            one(j * _L)
    else:
        U = 8

        @plsc.parallel_loop(0, nvec // U)
        def _(i):
            base = i * (U * _L)
            for u in range(U):
                one(base + u * _L)


def _do_row(in_hbm, out_hbm, row, n, k, xv_ref, kv_ref, tmp_ref):
    pltpu.sync_copy(in_hbm.at[pl.ds(row * n, n)], xv_ref.at[pl.ds(0, n)])
    cnt_neg = _keys_and_negcount(xv_ref, kv_ref, n, tmp_ref)
    ks = _select_threshold(kv_ref, n, k, tmp_ref, cnt_neg)
    _apply_mask(xv_ref, kv_ref, n, ks)
    pltpu.sync_copy(xv_ref.at[pl.ds(0, n)], out_hbm.at[pl.ds(row * n, n)])


def _sc_body(zh_hbm, zi_hbm, zl_hbm, oh_hbm, oi_hbm, ol_hbm, x0, x1, kv, tmp,
             sem_in, sem_out):
    wid = lax.axis_index("s") * 2 + lax.axis_index("c")
    n, k = _INTERMEDIATE, _K_INT
    rows = [wid, wid + _NW, wid + 2 * _NW]
    bufs = [x0, x1, x0]

    def compute(xv_ref, ks_row):
        cnt_neg = _keys_and_negcount(xv_ref, kv, n, tmp)
        ks = _select_threshold(kv, n, k, tmp, cnt_neg)
        _apply_mask(xv_ref, kv, n, ks)

    h_in0 = pltpu.async_copy(zi_hbm.at[rows[0]], x0.at[pl.ds(0, n)], sem_in)

    h_in0.wait()
    h_in1 = pltpu.async_copy(zi_hbm.at[rows[1]], x1.at[pl.ds(0, n)], sem_in)
    compute(x0, 0)
    h_out0 = pltpu.async_copy(x0.at[pl.ds(0, n)], oi_hbm.at[rows[0], 0, 0], sem_out)

    h_in1.wait()
    cnt_neg1 = _keys_and_negcount(x1, kv, n, tmp)
    h_out0.wait()

    @pl.when(wid < _N_LAYERS - 2 * _NW)
    def _():
        pltpu.async_copy(zi_hbm.at[rows[2]], x0.at[pl.ds(0, n)], sem_in).wait()

    ks1 = _select_threshold(kv, n, k, tmp, cnt_neg1)
    _apply_mask(x1, kv, n, ks1)
    h_out1 = pltpu.async_copy(x1.at[pl.ds(0, n)], oi_hbm.at[rows[1], 0, 0], sem_out)

    @pl.when(wid < _N_LAYERS - 2 * _NW)
    def _():
        compute(x0, 2)
        pltpu.async_copy(x0.at[pl.ds(0, n)], oi_hbm.at[rows[2], 0, 0], sem_out).wait()

    @pl.when(wid >= _NW // 2)
    def _():
        for j in range(5):
            row = (wid - _NW // 2) * 5 + j
            _do_row(zh_hbm, oh_hbm, row, _N_HEADS, _K_HEAD, x0, kv, tmp)

    @pl.when(wid == _NW - 1)
    def _():
        pltpu.sync_copy(zl_hbm, x0.at[pl.ds(0, _N_LAYERS)])
        cnt_neg = _keys_and_negcount(x0, kv, _N_LAYERS, tmp)
        ks = _select_threshold(kv, _N_LAYERS, _K_LAYER, tmp, cnt_neg)
        _apply_mask(x0, kv, _N_LAYERS, ks)
        pltpu.sync_copy(x0.at[pl.ds(0, _N_LAYERS)], ol_hbm)

    h_out1.wait()


def kernel(z_loga_head, z_loga_intermediate, z_loga_layer):
    mesh = plsc.VectorSubcoreMesh(
        core_axis_name="c", subcore_axis_name="s", num_cores=2, num_subcores=16
    )
    f = pl.kernel(
        _sc_body,
        out_type=(
            jax.ShapeDtypeStruct((_N_LAYERS * _N_HEADS,), jnp.float32),
            jax.ShapeDtypeStruct((_N_LAYERS, 1, 1, _INTERMEDIATE), jnp.float32),
            jax.ShapeDtypeStruct((_N_LAYERS,), jnp.float32),
        ),
        mesh=mesh,
        scratch_types=[
            pltpu.VMEM((_INTERMEDIATE,), jnp.float32),
            pltpu.VMEM((_INTERMEDIATE,), jnp.float32),
            pltpu.VMEM((_INTERMEDIATE,), jnp.int32),
            pltpu.VMEM((2 * _L,), jnp.int32),
            pltpu.SemaphoreType.DMA,
            pltpu.SemaphoreType.DMA,
        ],
    )
    zh, zi, zl = f(z_loga_head.reshape(-1), z_loga_intermediate, z_loga_layer)
    return (
        zh.reshape(_N_LAYERS, 1, _N_HEADS, 1, 1),
        zi,
        zl,
    )


# speedup vs baseline: 1.0046x; 1.0046x over previous
import jax
import jax.numpy as jnp
from jax import lax
from jax.experimental import pallas as pl
from jax.experimental.pallas import tpu as pltpu
from jax.experimental.pallas import tpu_sc as plsc

_N_LAYERS = 80
_N_HEADS = 64
_INTERMEDIATE = 28672
_K_HEAD = 16
_K_INT = 7168
_K_LAYER = 16
_SCALE = 1.2

_L = 16
_INT_MIN = -(2**31)
_M31 = 0x7FFFFFFF
_NW = 32


def _lane_total(x, tmp_ref):
    for s in (8, 4, 2, 1):
        tmp_ref[pl.ds(0, _L)] = x
        tmp_ref[pl.ds(_L, _L)] = x
        x = x + tmp_ref[pl.ds(s, _L)]
    return x


def _keys_and_negcount(xv_ref, kv_ref, n, tmp_ref):
    nvec = n // _L
    zero = jnp.zeros((_L,), jnp.int32)

    def one(base, a, u):
        x = xv_ref[pl.ds(base, _L)]
        iv = lax.bitcast_convert_type(x, jnp.int32)
        sh = iv >> 31
        kv_ref[pl.ds(base, _L)] = iv ^ (sh & _M31)
        a[u % 2] = a[u % 2] + sh
        return a

    if nvec <= 8:
        a = [zero, zero]
        for j in range(nvec):
            a = one(j * _L, a, j)
        neg = a[0] + a[1]
    else:
        U = 16

        @plsc.parallel_loop(0, nvec // U, carry=(zero, zero))
        def accs(i, a):
            a = list(a)
            base = i * (U * _L)
            for u in range(U):
                a = one(base + u * _L, a, u)
            return tuple(a)

        neg = accs[0] + accs[1]
    return -_lane_total(neg, tmp_ref)[0]


def _select_threshold(kv_ref, n, k, tmp_ref, cnt_neg):
    nvec = n // _L

    def bit_body(t, st):
        p, ks, done = st

        def active(_):
            b = 31 - t
            c = p | (jnp.int32(1) << b)
            cs = c ^ _INT_MIN
            if nvec <= 16:
                acc = jnp.zeros((_L,), jnp.int32)
                for j in range(nvec):
                    kvv = kv_ref[pl.ds(j * _L, _L)]
                    acc = acc + jnp.where(kvv < cs, 1, 0)
            else:
                U = 32
                A = 4
                zero = jnp.zeros((_L,), jnp.int32)

                @plsc.parallel_loop(0, nvec // U, carry=(zero,) * A)
                def accs(i, a):
                    a = list(a)
                    base = i * (U * _L)
                    for u in range(U):
                        kvv = kv_ref[pl.ds(base + u * _L, _L)]
                        a[u % A] = a[u % A] + jnp.where(kvv < cs, 1, 0)
                    return tuple(a)

                acc = (accs[0] + accs[1]) + (accs[2] + accs[3])
            cnt = _lane_total(acc, tmp_ref)[0]
            return (
                jnp.where(cnt >= k, p, c),
                jnp.where(cnt == k, cs - 1, ks),
                cnt == k,
            )

        def idle(_):
            return p, ks, done

        return lax.cond(jnp.logical_not(done), active, idle, 0)

    p0 = jnp.where(cnt_neg >= k, jnp.int32(0), jnp.int32(_INT_MIN))
    done0 = cnt_neg == k
    ks0 = jnp.where(done0, jnp.int32(-1), jnp.int32(0))
    p, ks, done = lax.fori_loop(1, 32, bit_body, (p0, ks0, done0))
    return jnp.where(done, ks, p ^ _INT_MIN)


def _apply_mask(xv_ref, kv_ref, n, ks):
    nvec = n // _L

    def one(j_base):
        x = xv_ref[pl.ds(j_base, _L)]
        kvv = kv_ref[pl.ds(j_base, _L)]
        s = 1.0 / (1.0 + jnp.exp(x * jnp.float32(-_SCALE)))
        xv_ref[pl.ds(j_base, _L)] = jnp.where(kvv <= ks, jnp.float32(0.0), s)

    if nvec <= 8:
        for j in range(nvec):
            one(j * _L)
    else:
        U = 16

        @plsc.parallel_loop(0, nvec // U)
        def _(i):
            base = i * (U * _L)
            for u in range(U):
                one(base + u * _L)


def _do_row(in_hbm, out_hbm, row, n, k, xv_ref, kv_ref, tmp_ref):
    pltpu.sync_copy(in_hbm.at[pl.ds(row * n, n)], xv_ref.at[pl.ds(0, n)])
    cnt_neg = _keys_and_negcount(xv_ref, kv_ref, n, tmp_ref)
    ks = _select_threshold(kv_ref, n, k, tmp_ref, cnt_neg)
    _apply_mask(xv_ref, kv_ref, n, ks)
    pltpu.sync_copy(xv_ref.at[pl.ds(0, n)], out_hbm.at[pl.ds(row * n, n)])


def _sc_body(zh_hbm, zi_hbm, zl_hbm, oh_hbm, oi_hbm, ol_hbm, x0, x1, kv, tmp,
             sem_in, sem_out):
    wid = lax.axis_index("s") * 2 + lax.axis_index("c")
    n, k = _INTERMEDIATE, _K_INT
    rows = [wid, wid + _NW, wid + 2 * _NW]
    bufs = [x0, x1, x0]

    def compute(xv_ref, ks_row):
        cnt_neg = _keys_and_negcount(xv_ref, kv, n, tmp)
        ks = _select_threshold(kv, n, k, tmp, cnt_neg)
        _apply_mask(xv_ref, kv, n, ks)

    h_in0 = pltpu.async_copy(zi_hbm.at[rows[0]], x0.at[pl.ds(0, n)], sem_in)

    h_in0.wait()
    h_in1 = pltpu.async_copy(zi_hbm.at[rows[1]], x1.at[pl.ds(0, n)], sem_in)
    compute(x0, 0)
    h_out0 = pltpu.async_copy(x0.at[pl.ds(0, n)], oi_hbm.at[rows[0], 0, 0], sem_out)

    h_in1.wait()
    cnt_neg1 = _keys_and_negcount(x1, kv, n, tmp)
    h_out0.wait()

    @pl.when(wid < _N_LAYERS - 2 * _NW)
    def _():
        pltpu.async_copy(zi_hbm.at[rows[2]], x0.at[pl.ds(0, n)], sem_in).wait()

    ks1 = _select_threshold(kv, n, k, tmp, cnt_neg1)
    _apply_mask(x1, kv, n, ks1)
    h_out1 = pltpu.async_copy(x1.at[pl.ds(0, n)], oi_hbm.at[rows[1], 0, 0], sem_out)

    @pl.when(wid < _N_LAYERS - 2 * _NW)
    def _():
        compute(x0, 2)
        pltpu.async_copy(x0.at[pl.ds(0, n)], oi_hbm.at[rows[2], 0, 0], sem_out).wait()

    @pl.when(wid >= _NW // 2)
    def _():
        for j in range(5):
            row = (wid - _NW // 2) * 5 + j
            _do_row(zh_hbm, oh_hbm, row, _N_HEADS, _K_HEAD, x0, kv, tmp)

    @pl.when(wid == _NW - 1)
    def _():
        pltpu.sync_copy(zl_hbm, x0.at[pl.ds(0, _N_LAYERS)])
        cnt_neg = _keys_and_negcount(x0, kv, _N_LAYERS, tmp)
        ks = _select_threshold(kv, _N_LAYERS, _K_LAYER, tmp, cnt_neg)
        _apply_mask(x0, kv, _N_LAYERS, ks)
        pltpu.sync_copy(x0.at[pl.ds(0, _N_LAYERS)], ol_hbm)

    h_out1.wait()


def kernel(z_loga_head, z_loga_intermediate, z_loga_layer):
    mesh = plsc.VectorSubcoreMesh(
        core_axis_name="c", subcore_axis_name="s", num_cores=2, num_subcores=16
    )
    f = pl.kernel(
        _sc_body,
        out_type=(
            jax.ShapeDtypeStruct((_N_LAYERS * _N_HEADS,), jnp.float32),
            jax.ShapeDtypeStruct((_N_LAYERS, 1, 1, _INTERMEDIATE), jnp.float32),
            jax.ShapeDtypeStruct((_N_LAYERS,), jnp.float32),
        ),
        mesh=mesh,
        scratch_types=[
            pltpu.VMEM((_INTERMEDIATE,), jnp.float32),
            pltpu.VMEM((_INTERMEDIATE,), jnp.float32),
            pltpu.VMEM((_INTERMEDIATE,), jnp.int32),
            pltpu.VMEM((2 * _L,), jnp.int32),
            pltpu.SemaphoreType.DMA,
            pltpu.SemaphoreType.DMA,
        ],
    )
    zh, zi, zl = f(z_loga_head.reshape(-1), z_loga_intermediate, z_loga_layer)
    return (
        zh.reshape(_N_LAYERS, 1, _N_HEADS, 1, 1),
        zi,
        zl,
    )


# speedup vs baseline: 1.0050x; 1.0004x over previous
import jax
import jax.numpy as jnp
from jax import lax
from jax.experimental import pallas as pl
from jax.experimental.pallas import tpu as pltpu
from jax.experimental.pallas import tpu_sc as plsc

_N_LAYERS = 80
_N_HEADS = 64
_INTERMEDIATE = 28672
_K_HEAD = 16
_K_INT = 7168
_K_LAYER = 16
_SCALE = 1.2

_L = 16
_INT_MIN = -(2**31)
_M31 = 0x7FFFFFFF
_NW = 32


def _lane_total(x, tmp_ref):
    for s in (8, 4, 2, 1):
        tmp_ref[pl.ds(0, _L)] = x
        tmp_ref[pl.ds(_L, _L)] = x
        x = x + tmp_ref[pl.ds(s, _L)]
    return x


def _keys_and_negcount(xv_ref, kv_ref, n, tmp_ref):
    nvec = n // _L
    zero = jnp.zeros((_L,), jnp.int32)

    def one(base, a, u):
        x = xv_ref[pl.ds(base, _L)]
        iv = lax.bitcast_convert_type(x, jnp.int32)
        sh = iv >> 31
        kv_ref[pl.ds(base, _L)] = iv ^ (sh & _M31)
        a[u % 2] = a[u % 2] + sh
        return a

    if nvec <= 8:
        a = [zero, zero]
        for j in range(nvec):
            a = one(j * _L, a, j)
        neg = a[0] + a[1]
    else:
        U = 16

        @plsc.parallel_loop(0, nvec // U, carry=(zero, zero))
        def accs(i, a):
            a = list(a)
            base = i * (U * _L)
            for u in range(U):
                a = one(base + u * _L, a, u)
            return tuple(a)

        neg = accs[0] + accs[1]
    return -_lane_total(neg, tmp_ref)[0]


def _select_threshold(kv_ref, n, k, tmp_ref, cnt_neg):
    nvec = n // _L

    def bit_body(t, st):
        p, ks, done = st

        def active(_):
            b = 31 - t
            c = p | (jnp.int32(1) << b)
            cs = c ^ _INT_MIN
            if nvec <= 16:
                acc = jnp.zeros((_L,), jnp.int32)
                for j in range(nvec):
                    kvv = kv_ref[pl.ds(j * _L, _L)]
                    acc = acc + jnp.where(kvv < cs, 1, 0)
            else:
                U = 32
                A = 4
                zero = jnp.zeros((_L,), jnp.int32)

                @plsc.parallel_loop(0, nvec // U, carry=(zero,) * A)
                def accs(i, a):
                    a = list(a)
                    base = i * (U * _L)
                    for u in range(U):
                        kvv = kv_ref[pl.ds(base + u * _L, _L)]
                        a[u % A] = a[u % A] + jnp.where(kvv < cs, 1, 0)
                    return tuple(a)

                acc = (accs[0] + accs[1]) + (accs[2] + accs[3])
            cnt = _lane_total(acc, tmp_ref)[0]
            return (
                jnp.where(cnt >= k, p, c),
                jnp.where(cnt == k, cs - 1, ks),
                cnt == k,
            )

        def idle(_):
            return p, ks, done

        return lax.cond(jnp.logical_not(done), active, idle, 0)

    p0 = jnp.where(cnt_neg >= k, jnp.int32(0), jnp.int32(_INT_MIN))
    done0 = cnt_neg == k
    ks0 = jnp.where(done0, jnp.int32(-1), jnp.int32(0))
    p, ks, done = lax.fori_loop(1, 32, bit_body, (p0, ks0, done0))
    return jnp.where(done, ks, p ^ _INT_MIN)


def _apply_mask(xv_ref, kv_ref, n, ks):
    nvec = n // _L

    def one(j_base):
        x = xv_ref[pl.ds(j_base, _L)]
        kvv = kv_ref[pl.ds(j_base, _L)]
        s = 1.0 / (1.0 + jnp.exp(x * jnp.float32(-_SCALE)))
        xv_ref[pl.ds(j_base, _L)] = jnp.where(kvv <= ks, jnp.float32(0.0), s)

    if nvec <= 8:
        for j in range(nvec):
            one(j * _L)
    else:
        U = 16

        @plsc.parallel_loop(0, nvec // U)
        def _(i):
            base = i * (U * _L)
            for u in range(U):
                one(base + u * _L)


def _do_row(in_hbm, out_hbm, row, n, k, xv_ref, kv_ref, tmp_ref):
    pltpu.sync_copy(in_hbm.at[pl.ds(row * n, n)], xv_ref.at[pl.ds(0, n)])
    cnt_neg = _keys_and_negcount(xv_ref, kv_ref, n, tmp_ref)
    ks = _select_threshold(kv_ref, n, k, tmp_ref, cnt_neg)
    _apply_mask(xv_ref, kv_ref, n, ks)
    pltpu.sync_copy(xv_ref.at[pl.ds(0, n)], out_hbm.at[pl.ds(row * n, n)])


def _sc_body(zh_hbm, zi_hbm, zl_hbm, oh_hbm, oi_hbm, ol_hbm, x0, x1, kv, tmp,
             sem_in, sem_out):
    wid = lax.axis_index("s") * 2 + lax.axis_index("c")
    n, k = _INTERMEDIATE, _K_INT
    rows = [wid, wid + _NW, wid + 2 * _NW]

    def compute(xv_ref):
        cnt_neg = _keys_and_negcount(xv_ref, kv, n, tmp)
        ks = _select_threshold(kv, n, k, tmp, cnt_neg)
        _apply_mask(xv_ref, kv, n, ks)

    h_in0 = pltpu.async_copy(zi_hbm.at[rows[0]], x0.at[pl.ds(0, n)], sem_in)

    h_in0.wait()
    h_in1 = pltpu.async_copy(zi_hbm.at[rows[1]], x1.at[pl.ds(0, n)], sem_in)
    compute(x0)
    h_out0 = pltpu.async_copy(x0.at[pl.ds(0, n)], oi_hbm.at[rows[0], 0, 0], sem_out)

    h_in1.wait()
    cnt_neg1 = _keys_and_negcount(x1, kv, n, tmp)
    h_out0.wait()

    @pl.when(wid < _N_LAYERS - 2 * _NW)
    def _():
        pltpu.async_copy(zi_hbm.at[rows[2]], x0.at[pl.ds(0, n)], sem_in).wait()

    ks1 = _select_threshold(kv, n, k, tmp, cnt_neg1)
    _apply_mask(x1, kv, n, ks1)
    h_out1 = pltpu.async_copy(x1.at[pl.ds(0, n)], oi_hbm.at[rows[1], 0, 0], sem_out)

    @pl.when(wid < _N_LAYERS - 2 * _NW)
    def _():
        compute(x0)
        pltpu.async_copy(x0.at[pl.ds(0, n)], oi_hbm.at[rows[2], 0, 0], sem_out).wait()

    @pl.when(wid >= _NW // 2)
    def _():
        for j in range(5):
            row = (wid - _NW // 2) * 5 + j
            _do_row(zh_hbm, oh_hbm, row, _N_HEADS, _K_HEAD, x0, kv, tmp)

    @pl.when(wid == _NW - 1)
    def _():
        pltpu.sync_copy(zl_hbm, x0.at[pl.ds(0, _N_LAYERS)])
        cnt_neg = _keys_and_negcount(x0, kv, _N_LAYERS, tmp)
        ks = _select_threshold(kv, _N_LAYERS, _K_LAYER, tmp, cnt_neg)
        _apply_mask(x0, kv, _N_LAYERS, ks)
        pltpu.sync_copy(x0.at[pl.ds(0, _N_LAYERS)], ol_hbm)

    h_out1.wait()


def kernel(z_loga_head, z_loga_intermediate, z_loga_layer):
    mesh = plsc.VectorSubcoreMesh(
        core_axis_name="c", subcore_axis_name="s", num_cores=2, num_subcores=16
    )
    f = pl.kernel(
        _sc_body,
        out_type=(
            jax.ShapeDtypeStruct((_N_LAYERS * _N_HEADS,), jnp.float32),
            jax.ShapeDtypeStruct((_N_LAYERS, 1, 1, _INTERMEDIATE), jnp.float32),
            jax.ShapeDtypeStruct((_N_LAYERS,), jnp.float32),
        ),
        mesh=mesh,
        scratch_types=[
            pltpu.VMEM((_INTERMEDIATE,), jnp.float32),
            pltpu.VMEM((_INTERMEDIATE,), jnp.float32),
            pltpu.VMEM((_INTERMEDIATE,), jnp.int32),
            pltpu.VMEM((2 * _L,), jnp.int32),
            pltpu.SemaphoreType.DMA,
            pltpu.SemaphoreType.DMA,
        ],
    )
    zh, zi, zl = f(z_loga_head.reshape(-1), z_loga_intermediate, z_loga_layer)
    return (
        zh.reshape(_N_LAYERS, 1, _N_HEADS, 1, 1),
        zi,
        zl,
    )
